# baseline (speedup 1.0000x reference)
import functools

import jax
import jax.numpy as jnp
from jax import lax
from jax.experimental import pallas as pl
from jax.experimental.pallas import tpu as pltpu
from jax.experimental.pallas import tpu_sc as plsc

_PH = _PW = 7
_SR = 2
_SCALE = 0.25
_L = 16

_info = plsc.get_sparse_core_info()
_NC, _NS = _info.num_cores, _info.num_subcores
_NW = _NC * _NS


def _roi_align_sc(feat, rois_flat, H, W, C, R):
    NB = _PH * _PW
    RL = _PW * _L
    rpw = R // _NW
    mesh = plsc.VectorSubcoreMesh(core_axis_name="c", subcore_axis_name="s")

    @functools.partial(
        pl.kernel,
        mesh=mesh,
        out_type=jax.ShapeDtypeStruct((R * NB * C,), jnp.float32),
        scratch_types=[
            pltpu.VMEM((_L,), jnp.float32),
            pltpu.VMEM((RL,), jnp.int32),
            pltpu.VMEM((RL,), jnp.int32),
            pltpu.VMEM((RL,), jnp.float32),
            pltpu.VMEM((RL,), jnp.float32),
            pltpu.VMEM((RL, C // 2), jnp.int32),
            pltpu.VMEM((RL, C // 2), jnp.int32),
            pltpu.VMEM((NB * C,), jnp.float32),
            pltpu.SemaphoreType.DMA,
            pltpu.SemaphoreType.DMA,
            pltpu.SemaphoreType.DMA,
        ],
    )
    def run(feat_hbm, rois_hbm, out_hbm, roiv, idx0, idx1, wb0, wb1,
            rows0, rows1, outb, sem0, sem1, osem):
        wid = lax.axis_index("s") * _NC + lax.axis_index("c")
        base = wid * rpw
        lane = lax.iota(jnp.int32, _L)
        lsy = ((lane >> 3) & 1).astype(jnp.float32)
        lsx = ((lane >> 2) & 1).astype(jnp.float32)
        nyi = (lane >> 1) & 1
        nxi = lane & 1
        nysel = nyi == 1
        nxsel = nxi == 1

        def axis_vals(pos_f, lo, bsz, size):
            g = lo + (pos_f + 0.5) * (1.0 / _SR) * bsz
            v = jnp.where((g >= -1.0) & (g <= float(size)), 1.0, 0.0)
            yc = jnp.maximum(g, 0.0)
            ilf = yc.astype(jnp.int32)
            cond = ilf >= size - 1
            yc = jnp.where(cond, float(size - 1), yc)
            ilf = jnp.where(cond, size - 1, ilf)
            lw = yc - ilf.astype(jnp.float32)
            hw = 1.0 - lw
            return ilf, hw * v, lw * v

        def load_roi(rc):
            pltpu.sync_copy(rois_hbm.at[pl.ds((base + rc) * _L, _L)], roiv)

        def build_row(py, idxb, wbuf):
            rv = roiv[...]

            def lanebc(i):
                return jnp.broadcast_to(lax.slice(rv, (i,), (i + 1,)), (_L,))

            bi = lanebc(0).astype(jnp.int32)
            x1 = lanebc(1) * _SCALE - 0.5
            y1 = lanebc(2) * _SCALE - 0.5
            x2 = lanebc(3) * _SCALE - 0.5
            y2 = lanebc(4) * _SCALE - 0.5
            bh = (y2 - y1) * (1.0 / _PH)
            bw = (x2 - x1) * (1.0 / _PW)
            boff = bi * (H * W)
            t_f = jnp.broadcast_to(py, (_L,)).astype(jnp.float32) * _SR + lsy
            ylo, why, wly = axis_vals(t_f, y1, bh, H)
            yv = jnp.minimum(ylo + nyi, H - 1)
            wy = jnp.where(nysel, wly, why) * (1.0 / (_SR * _SR))
            ybase = boff + yv * W

            def build(px, _):
                u_f = (jnp.broadcast_to(px, (_L,)).astype(jnp.float32)
                       * _SR + lsx)
                xlo, whx, wlx = axis_vals(u_f, x1, bw, W)
                xv = jnp.minimum(xlo + nxi, W - 1)
                wx = jnp.where(nxsel, wlx, whx)
                idxb[pl.ds(px * _L, _L)] = ybase + xv
                wbuf[pl.ds(px * _L, _L)] = wy * wx
                return 0

            lax.fori_loop(0, _PW, build, 0)

        def gather(idxb, rows, sem):
            return pltpu.make_async_copy(feat_hbm.at[idxb], rows, sem)

        def out_copy(rc):
            return pltpu.make_async_copy(
                outb, out_hbm.at[pl.ds((base + rc) * NB * C, NB * C)], osem)

        def process_row(py, rows, wbuf):
            @plsc.parallel_loop(0, _PW)
            def px_body(px):
                ebase = px * _L
                wv = wbuf[pl.ds(ebase, _L)]
                accs = [None] * (C // _L)
                for e in range(_L):
                    wb = jnp.broadcast_to(
                        lax.slice(wv, (e,), (e + 1,)), (_L,))
                    half = C // (2 * _L)
                    for k in range(half):
                        pair = rows[ebase + e, pl.ds(k * _L, _L)]
                        a = lax.bitcast_convert_type(pair << 16,
                                                     jnp.float32)
                        b = lax.bitcast_convert_type(pair, jnp.float32)
                        ta, tb = wb * a, wb * b
                        if e == 0:
                            accs[k], accs[half + k] = ta, tb
                        else:
                            accs[k] = accs[k] + ta
                            accs[half + k] = accs[half + k] + tb
                obase = (py * _PW + px) * C
                for k in range(C // _L):
                    outb[pl.ds(obase + k * _L, _L)] = accs[k]

        load_roi(0)
        build_row(0, idx0, wb0)
        gather(idx0, rows0, sem0).start()

        def step(s, carry):
            py, rc = carry
            last = py == _PH - 1
            havenext = jnp.logical_or(jnp.logical_not(last), rc + 1 < rpw)
            py1 = jnp.where(last, 0, py + 1)
            rc1 = jnp.where(last, rc + 1, rc)

            def prep(idxn, wbn, rowsn, semn):
                @pl.when(havenext)
                def _():
                    @pl.when(last)
                    def _():
                        load_roi(rc1)

                    build_row(py1, idxn, wbn)
                    gather(idxn, rowsn, semn).start()

            def stage(idxc, wbc, rowsc, semc, idxn, wbn, rowsn, semn):
                gather(idxc, rowsc, semc).wait()
                prep(idxn, wbn, rowsn, semn)
                @pl.when(jnp.logical_and(py == 0, rc > 0))
                def _():
                    out_copy(rc - 1).wait()

                process_row(py, rowsc, wbc)

                @pl.when(last)
                def _():
                    out_copy(rc).start()

            @pl.when((s & 1) == 0)
            def _():
                stage(idx0, wb0, rows0, sem0, idx1, wb1, rows1, sem1)

            @pl.when((s & 1) == 1)
            def _():
                stage(idx1, wb1, rows1, sem1, idx0, wb0, rows0, sem0)

            return py1, rc1

        lax.fori_loop(0, rpw * _PH, step, (jnp.int32(0), jnp.int32(0)))
        out_copy(rpw - 1).wait()

    return run(feat, rois_flat)


def kernel(input, rois):
    N, C, H, W = input.shape
    R = rois.shape[0]
    P = N * H * W

    def rne(b):
        return (b + 0x7FFF + ((b >> 16) & 1)) >> 16

    bits = lax.bitcast_convert_type(input, jnp.int32)
    packed = (rne(bits[:, :C // 2]) & 0xFFFF) | (rne(bits[:, C // 2:]) << 16)
    feat = jnp.transpose(packed, (0, 2, 3, 1)).reshape(P, C // 2)
    rois_flat = jnp.concatenate(
        [rois, jnp.zeros((R, _L - rois.shape[1]), rois.dtype)],
        axis=1).reshape(-1)
    out = _roi_align_sc(feat, rois_flat, H, W, C, R)
    return jnp.transpose(out.reshape(R, _PH * _PW, C),
                         (0, 2, 1)).reshape(R, C, _PH, _PW)


# speedup vs baseline: 1.7356x; 1.7356x over previous
import functools

import jax
import jax.numpy as jnp
from jax import lax
from jax.experimental import pallas as pl
from jax.experimental.pallas import tpu as pltpu
from jax.experimental.pallas import tpu_sc as plsc

_PH = _PW = 7
_SR = 2
_SCALE = 0.25
_L = 16

_info = plsc.get_sparse_core_info()
_NC, _NS = _info.num_cores, _info.num_subcores
_NW = _NC * _NS


def _roi_align_sc(feat, rois_flat, H, W, C, R):
    NB = _PH * _PW
    RL = _PW * _L
    rpw = R // _NW
    mesh = plsc.VectorSubcoreMesh(core_axis_name="c", subcore_axis_name="s")

    @functools.partial(
        pl.kernel,
        mesh=mesh,
        out_type=jax.ShapeDtypeStruct((R * NB * C,), jnp.float32),
        scratch_types=[
            pltpu.VMEM((_L,), jnp.float32),
            pltpu.VMEM((RL,), jnp.int32),
            pltpu.VMEM((RL,), jnp.int32),
            pltpu.VMEM((RL,), jnp.float32),
            pltpu.VMEM((RL,), jnp.float32),
            pltpu.VMEM((RL, C // 2), jnp.int32),
            pltpu.VMEM((RL, C // 2), jnp.int32),
            pltpu.VMEM((NB * C,), jnp.float32),
            pltpu.SemaphoreType.DMA,
            pltpu.SemaphoreType.DMA,
            pltpu.SemaphoreType.DMA,
        ],
    )
    def run(feat_hbm, rois_hbm, out_hbm, roiv, idx0, idx1, wb0, wb1,
            rows0, rows1, outb, sem0, sem1, osem):
        wid = lax.axis_index("s") * _NC + lax.axis_index("c")
        base = wid * rpw
        lane = lax.iota(jnp.int32, _L)
        lsy = ((lane >> 3) & 1).astype(jnp.float32)
        lsx = ((lane >> 2) & 1).astype(jnp.float32)
        nyi = (lane >> 1) & 1
        nxi = lane & 1
        nysel = nyi == 1
        nxsel = nxi == 1

        def axis_vals(pos_f, lo, bsz, size):
            g = lo + (pos_f + 0.5) * (1.0 / _SR) * bsz
            v = jnp.where((g >= -1.0) & (g <= float(size)), 1.0, 0.0)
            yc = jnp.maximum(g, 0.0)
            ilf = yc.astype(jnp.int32)
            cond = ilf >= size - 1
            yc = jnp.where(cond, float(size - 1), yc)
            ilf = jnp.where(cond, size - 1, ilf)
            lw = yc - ilf.astype(jnp.float32)
            hw = 1.0 - lw
            return ilf, hw * v, lw * v

        def load_roi(rc):
            pltpu.sync_copy(rois_hbm.at[pl.ds((base + rc) * _L, _L)], roiv)

        def build_row(py, idxb, wbuf):
            rv = roiv[...]

            def lanebc(i):
                return jnp.broadcast_to(lax.slice(rv, (i,), (i + 1,)), (_L,))

            bi = lanebc(0).astype(jnp.int32)
            x1 = lanebc(1) * _SCALE - 0.5
            y1 = lanebc(2) * _SCALE - 0.5
            x2 = lanebc(3) * _SCALE - 0.5
            y2 = lanebc(4) * _SCALE - 0.5
            bh = (y2 - y1) * (1.0 / _PH)
            bw = (x2 - x1) * (1.0 / _PW)
            boff = bi * (H * W)
            t_f = jnp.broadcast_to(py, (_L,)).astype(jnp.float32) * _SR + lsy
            ylo, why, wly = axis_vals(t_f, y1, bh, H)
            yv = jnp.minimum(ylo + nyi, H - 1)
            wy = jnp.where(nysel, wly, why) * (1.0 / (_SR * _SR))
            ybase = boff + yv * W

            def build(px, _):
                u_f = (jnp.broadcast_to(px, (_L,)).astype(jnp.float32)
                       * _SR + lsx)
                xlo, whx, wlx = axis_vals(u_f, x1, bw, W)
                xv = jnp.minimum(xlo + nxi, W - 1)
                wx = jnp.where(nxsel, wlx, whx)
                idxb[pl.ds(px * _L, _L)] = ybase + xv
                wbuf[pl.ds(px * _L, _L)] = wy * wx
                return 0

            lax.fori_loop(0, _PW, build, 0)

        def gather(idxb, rows, sem):
            return pltpu.make_async_copy(feat_hbm.at[idxb], rows, sem)

        def out_copy(rc):
            return pltpu.make_async_copy(
                outb, out_hbm.at[pl.ds((base + rc) * NB * C, NB * C)], osem)

        def process_row(py, rows, wbuf):
            def px_body(px, _):
                ebase = px * _L
                wv = wbuf[pl.ds(ebase, _L)]
                accs = [None] * (C // _L)
                for e in range(_L):
                    wb = jnp.broadcast_to(
                        lax.slice(wv, (e,), (e + 1,)), (_L,))
                    half = C // (2 * _L)
                    for k in range(half):
                        pair = rows[ebase + e, pl.ds(k * _L, _L)]
                        a = lax.bitcast_convert_type(pair << 16,
                                                     jnp.float32)
                        b = lax.bitcast_convert_type(pair, jnp.float32)
                        ta, tb = wb * a, wb * b
                        if e == 0:
                            accs[k], accs[half + k] = ta, tb
                        else:
                            accs[k] = accs[k] + ta
                            accs[half + k] = accs[half + k] + tb
                obase = (py * _PW + px) * C
                for k in range(C // _L):
                    outb[pl.ds(obase + k * _L, _L)] = accs[k]
                return 0

            lax.fori_loop(0, _PW, px_body, 0)

        load_roi(0)
        build_row(0, idx0, wb0)
        gather(idx0, rows0, sem0).start()

        def step(s, carry):
            py, rc = carry
            last = py == _PH - 1
            havenext = jnp.logical_or(jnp.logical_not(last), rc + 1 < rpw)
            py1 = jnp.where(last, 0, py + 1)
            rc1 = jnp.where(last, rc + 1, rc)

            def prep(idxn, wbn, rowsn, semn):
                @pl.when(havenext)
                def _():
                    @pl.when(last)
                    def _():
                        load_roi(rc1)

                    build_row(py1, idxn, wbn)
                    gather(idxn, rowsn, semn).start()

            def stage(idxc, wbc, rowsc, semc, idxn, wbn, rowsn, semn):
                gather(idxc, rowsc, semc).wait()
                prep(idxn, wbn, rowsn, semn)
                @pl.when(jnp.logical_and(py == 0, rc > 0))
                def _():
                    out_copy(rc - 1).wait()

                process_row(py, rowsc, wbc)

                @pl.when(last)
                def _():
                    out_copy(rc).start()

            @pl.when((s & 1) == 0)
            def _():
                stage(idx0, wb0, rows0, sem0, idx1, wb1, rows1, sem1)

            @pl.when((s & 1) == 1)
            def _():
                stage(idx1, wb1, rows1, sem1, idx0, wb0, rows0, sem0)

            return py1, rc1

        lax.fori_loop(0, rpw * _PH, step, (jnp.int32(0), jnp.int32(0)))
        out_copy(rpw - 1).wait()

    return run(feat, rois_flat)


def kernel(input, rois):
    N, C, H, W = input.shape
    R = rois.shape[0]
    P = N * H * W

    def rne(b):
        return (b + 0x7FFF + ((b >> 16) & 1)) >> 16

    bits = lax.bitcast_convert_type(input, jnp.int32)
    packed = (rne(bits[:, :C // 2]) & 0xFFFF) | (rne(bits[:, C // 2:]) << 16)
    feat = jnp.transpose(packed, (0, 2, 3, 1)).reshape(P, C // 2)
    rois_flat = jnp.concatenate(
        [rois, jnp.zeros((R, _L - rois.shape[1]), rois.dtype)],
        axis=1).reshape(-1)
    out = _roi_align_sc(feat, rois_flat, H, W, C, R)
    return jnp.transpose(out.reshape(R, _PH * _PW, C),
                         (0, 2, 1)).reshape(R, C, _PH, _PW)


# speedup vs baseline: 2.0496x; 1.1810x over previous
import functools

import jax
import jax.numpy as jnp
from jax import lax
from jax.experimental import pallas as pl
from jax.experimental.pallas import tpu as pltpu
from jax.experimental.pallas import tpu_sc as plsc

_PH = _PW = 7
_SR = 2
_SCALE = 0.25
_L = 16

_info = plsc.get_sparse_core_info()
_NC, _NS = _info.num_cores, _info.num_subcores
_NW = _NC * _NS


def _roi_align_sc(feat, rois_flat, H, W, C, R):
    NB = _PH * _PW
    RL = _PW * _L
    rpw = R // _NW
    mesh = plsc.VectorSubcoreMesh(core_axis_name="c", subcore_axis_name="s")

    @functools.partial(
        pl.kernel,
        mesh=mesh,
        out_type=jax.ShapeDtypeStruct((R * NB * C,), jnp.float32),
        scratch_types=[
            pltpu.VMEM((_L,), jnp.float32),
            pltpu.VMEM((RL,), jnp.int32),
            pltpu.VMEM((RL,), jnp.int32),
            pltpu.VMEM((RL,), jnp.float32),
            pltpu.VMEM((RL,), jnp.float32),
            pltpu.VMEM((RL, C // 2), jnp.int32),
            pltpu.VMEM((RL, C // 2), jnp.int32),
            pltpu.VMEM((NB * C,), jnp.float32),
            pltpu.SemaphoreType.DMA,
            pltpu.SemaphoreType.DMA,
            pltpu.SemaphoreType.DMA,
        ],
    )
    def run(feat_hbm, rois_hbm, out_hbm, roiv, idx0, idx1, wb0, wb1,
            rows0, rows1, outb, sem0, sem1, osem):
        wid = lax.axis_index("s") * _NC + lax.axis_index("c")
        base = wid * rpw
        lane = lax.iota(jnp.int32, _L)
        lsy = ((lane >> 3) & 1).astype(jnp.float32)
        lsx = ((lane >> 2) & 1).astype(jnp.float32)
        nyi = (lane >> 1) & 1
        nxi = lane & 1
        nysel = nyi == 1
        nxsel = nxi == 1

        def axis_vals(pos_f, lo, bsz, size):
            g = lo + (pos_f + 0.5) * (1.0 / _SR) * bsz
            v = jnp.where((g >= -1.0) & (g <= float(size)), 1.0, 0.0)
            yc = jnp.maximum(g, 0.0)
            ilf = yc.astype(jnp.int32)
            cond = ilf >= size - 1
            yc = jnp.where(cond, float(size - 1), yc)
            ilf = jnp.where(cond, size - 1, ilf)
            lw = yc - ilf.astype(jnp.float32)
            hw = 1.0 - lw
            return ilf, hw * v, lw * v

        def load_roi(rc):
            pltpu.sync_copy(rois_hbm.at[pl.ds((base + rc) * _L, _L)], roiv)

        def build_row(py, idxb, wbuf):
            rv = roiv[...]

            def lanebc(i):
                return jnp.broadcast_to(lax.slice(rv, (i,), (i + 1,)), (_L,))

            bi = lanebc(0).astype(jnp.int32)
            x1 = lanebc(1) * _SCALE - 0.5
            y1 = lanebc(2) * _SCALE - 0.5
            x2 = lanebc(3) * _SCALE - 0.5
            y2 = lanebc(4) * _SCALE - 0.5
            bh = (y2 - y1) * (1.0 / _PH)
            bw = (x2 - x1) * (1.0 / _PW)
            boff = bi * (H * W)
            t_f = jnp.broadcast_to(py, (_L,)).astype(jnp.float32) * _SR + lsy
            ylo, why, wly = axis_vals(t_f, y1, bh, H)
            yv = jnp.minimum(ylo + nyi, H - 1)
            wy = jnp.where(nysel, wly, why) * (1.0 / (_SR * _SR))
            ybase = boff + yv * W

            def build(px, _):
                u_f = (jnp.broadcast_to(px, (_L,)).astype(jnp.float32)
                       * _SR + lsx)
                xlo, whx, wlx = axis_vals(u_f, x1, bw, W)
                xv = jnp.minimum(xlo + nxi, W - 1)
                wx = jnp.where(nxsel, wlx, whx)
                idxb[pl.ds(px * _L, _L)] = ybase + xv
                wbuf[pl.ds(px * _L, _L)] = wy * wx
                return 0

            lax.fori_loop(0, _PW, build, 0)

        def gather(idxb, rows, sem):
            return pltpu.make_async_copy(feat_hbm.at[idxb], rows, sem)

        def out_copy(rc):
            return pltpu.make_async_copy(
                outb, out_hbm.at[pl.ds((base + rc) * NB * C, NB * C)], osem)

        def process_row(py, rows, wbuf):
            def px_body(px, _):
                ebase = px * _L
                wv = wbuf[pl.ds(ebase, _L)]
                accs = [None] * (C // _L)
                for e in range(_L):
                    wb = jnp.broadcast_to(
                        lax.slice(wv, (e,), (e + 1,)), (_L,))
                    half = C // (2 * _L)
                    for k in range(half):
                        pair = rows[ebase + e, pl.ds(k * _L, _L)]
                        a = lax.bitcast_convert_type(pair << 16,
                                                     jnp.float32)
                        b = lax.bitcast_convert_type(pair, jnp.float32)
                        ta, tb = wb * a, wb * b
                        if e == 0:
                            accs[k], accs[half + k] = ta, tb
                        else:
                            accs[k] = accs[k] + ta
                            accs[half + k] = accs[half + k] + tb
                obase = (py * _PW + px) * C
                for k in range(C // _L):
                    outb[pl.ds(obase + k * _L, _L)] = accs[k]
                return 0

            lax.fori_loop(0, _PW, px_body, 0)

        load_roi(0)
        build_row(0, idx0, wb0)
        gather(idx0, rows0, sem0).start()

        def step(s, carry):
            py, rc = carry
            last = py == _PH - 1
            havenext = jnp.logical_or(jnp.logical_not(last), rc + 1 < rpw)
            py1 = jnp.where(last, 0, py + 1)
            rc1 = jnp.where(last, rc + 1, rc)

            def prep(idxn, wbn, rowsn, semn):
                @pl.when(havenext)
                def _():
                    @pl.when(last)
                    def _():
                        load_roi(rc1)

                    build_row(py1, idxn, wbn)
                    gather(idxn, rowsn, semn).start()

            def stage(idxc, wbc, rowsc, semc, idxn, wbn, rowsn, semn):
                gather(idxc, rowsc, semc).wait()
                prep(idxn, wbn, rowsn, semn)
                @pl.when(jnp.logical_and(py == 0, rc > 0))
                def _():
                    out_copy(rc - 1).wait()

                process_row(py, rowsc, wbc)

                @pl.when(last)
                def _():
                    out_copy(rc).start()

            @pl.when((s & 1) == 0)
            def _():
                stage(idx0, wb0, rows0, sem0, idx1, wb1, rows1, sem1)

            @pl.when((s & 1) == 1)
            def _():
                stage(idx1, wb1, rows1, sem1, idx0, wb0, rows0, sem0)

            return py1, rc1

        lax.fori_loop(0, rpw * _PH, step, (jnp.int32(0), jnp.int32(0)))
        out_copy(rpw - 1).wait()

    return run(feat, rois_flat)


def kernel(input, rois):
    N, C, H, W = input.shape
    R = rois.shape[0]
    P = N * H * W

    def rne(b):
        return (b + 0x7FFF + ((b >> 16) & 1)) >> 16

    lo = lax.bitcast_convert_type(input[:, :C // 2], jnp.int32)
    hi = lax.bitcast_convert_type(input[:, C // 2:], jnp.int32)
    packed = (rne(lo) & 0xFFFF) | (rne(hi) << 16)
    feat = jnp.transpose(packed, (0, 2, 3, 1)).reshape(P, C // 2)
    rois_flat = jnp.concatenate(
        [rois, jnp.zeros((R, _L - rois.shape[1]), rois.dtype)],
        axis=1).reshape(-1)
    out = _roi_align_sc(feat, rois_flat, H, W, C, R)
    return jnp.transpose(out.reshape(R, _PH * _PW, C),
                         (0, 2, 1)).reshape(R, C, _PH, _PW)
